# baseline (speedup 1.0000x reference)
import jax
import jax.numpy as jnp
from jax import lax
from jax.experimental import pallas as pl
from jax.experimental.pallas import tpu as pltpu
from jax.experimental.pallas import tpu_sc as plsc

B = 32
DEG = 32
E = 2 * DEG
T = B * E
N_REL = 2000
NP = 2048
N_ENT = 10000
REL_DIM = 384
DIM = 256
TOPK = 8
LAYERS = 2
EPS = 1e-5
NC, NS = 2, 16
GROUP = 16
NGRP = E // GROUP
TBLK = 256
SEL_W = GROUP * TOPK


def _k1_body(rs_ref, wp_ref, bp_ref, ar_ref, an_ref,
             proj_ref, g_ref, arp_ref, anp_ref):
    p = jnp.dot(rs_ref[...], wp_ref[...],
                preferred_element_type=jnp.float32) + bp_ref[...]
    p = jnp.concatenate([p, jnp.zeros((NP - N_REL, DIM), jnp.float32)], axis=0)
    proj_ref[...] = p
    norm = jnp.sqrt(jnp.sum(p * p, axis=-1, keepdims=True))
    nhat = p / jnp.maximum(norm, 1e-12)
    g_ref[...] = lax.dot_general(nhat, nhat, (((1,), (1,)), ((), ())),
                                 preferred_element_type=jnp.float32)
    zpad = jnp.zeros((N_ENT, 128 - DEG), jnp.int32)
    arp_ref[...] = jnp.concatenate([ar_ref[...], zpad], axis=1)
    anp_ref[...] = jnp.concatenate([an_ref[...], zpad], axis=1)


def _run_k1(rel_semantics, w_proj, b_proj, adj_rel, adj_nbr):
    return pl.pallas_call(
        _k1_body,
        out_shape=(jax.ShapeDtypeStruct((NP, DIM), jnp.float32),
                   jax.ShapeDtypeStruct((NP, NP), jnp.float32),
                   jax.ShapeDtypeStruct((N_ENT, 128), jnp.int32),
                   jax.ShapeDtypeStruct((N_ENT, 128), jnp.int32)),
    )(rel_semantics, w_proj, b_proj.reshape(1, DIM), adj_rel, adj_nbr)


def _topk_step(j, base, nrel, sims, sel_ref, iota16, neg_inf):
    jj = j - base
    nid0 = nrel[j, pl.ds(0, 16)]
    nid1 = nrel[j, pl.ds(16, 16)]
    s0 = sims[jj, pl.ds(0, 16)]
    s1 = sims[jj, pl.ds(16, 16)]
    for it in range(TOPK):
        m = jnp.max(jnp.maximum(s0, s1))
        eq0 = s0 == m
        eq1 = s1 == m
        c0 = plsc.all_reduce_population_count(eq0)
        f0 = plsc.all_reduce_ffs(eq0)
        f1 = plsc.all_reduce_ffs(eq1)
        use0 = c0 > 0
        cand0 = nid0.at[f0].get(mode="promise_in_bounds")
        cand1 = nid1.at[f1].get(mode="promise_in_bounds")
        sel_id = jnp.where(use0, cand0, cand1)
        plsc.store_scatter(sel_ref,
                           [jnp.full((16,), jj * TOPK + it, jnp.int32)],
                           sel_id, mask=iota16 == 0)
        s0 = jnp.where(use0 & (iota16 == f0), neg_inf, s0)
        s1 = jnp.where((~use0) & (iota16 == f1), neg_inf, s1)
    return 0


def _sc_body(ents_hbm, adj_rel_hbm, adj_nbr_hbm, g_hbm, proj_hbm,
             states0_hbm, sel_hbm,
             ents_v, pidx, hrel, hnbr, nrel, relvec, ga, gb, sims, sel_v,
             sem_m, sem_ga, sem_gb):
    b = lax.axis_index("s") * NC + lax.axis_index("c")
    iota16 = lax.iota(jnp.int32, 16)
    neg_inf = jnp.float32(-jnp.inf)
    grows = (ga, gb)
    sem_g = (sem_ga, sem_gb)

    pltpu.sync_copy(ents_hbm, ents_v)
    h_spl = plsc.load_gather(ents_v, [jnp.full((16,), b, jnp.int32)])
    t_spl = plsc.load_gather(ents_v, [jnp.full((16,), b + B, jnp.int32)])
    pidx[...] = jnp.where(iota16 == 0, h_spl, t_spl)
    cp_r = pltpu.async_copy(adj_rel_hbm.at[pidx], hrel, sem_m)
    cp_n = pltpu.async_copy(adj_nbr_hbm.at[pidx], hnbr, sem_m)
    cp_r.wait()
    cp_n.wait()

    cps = []
    for h in range(2):
        cps.append(pltpu.async_copy(adj_rel_hbm.at[hnbr.at[h, pl.ds(0, DEG)]],
                                    nrel.at[pl.ds(DEG * h, DEG)], sem_m))
        cps.append(pltpu.async_copy(proj_hbm.at[hrel.at[h, pl.ds(0, DEG)]],
                                    relvec.at[pl.ds(DEG * h, DEG)], sem_m))
    pltpu.async_copy(g_hbm.at[hrel.at[0, pl.ds(0, GROUP)]], ga, sem_ga)
    for cp in cps:
        cp.wait()
    pltpu.sync_copy(relvec, states0_hbm.at[pl.ds(b * E, E)])

    for g in range(NGRP):
        cur = g % 2
        pltpu.make_async_copy(
            g_hbm.at[hrel.at[g // 2, pl.ds((g % 2) * GROUP, GROUP)]],
            grows[cur], sem_g[cur]).wait()
        if g + 1 < NGRP:
            g2 = g + 1
            pltpu.async_copy(
                g_hbm.at[hrel.at[g2 // 2, pl.ds((g2 % 2) * GROUP, GROUP)]],
                grows[g2 % 2], sem_g[g2 % 2])
        for j in range(GROUP):
            e = GROUP * g + j
            rowj = jnp.full((16,), j, jnp.int32)
            sims[j, pl.ds(0, 16)] = plsc.load_gather(
                grows[cur], [rowj, nrel[e, pl.ds(0, 16)]])
            sims[j, pl.ds(16, 16)] = plsc.load_gather(
                grows[cur], [rowj, nrel[e, pl.ds(16, 16)]])
        base = GROUP * g
        lax.fori_loop(base, base + GROUP,
                      lambda j, u: _topk_step(j, base, nrel, sims,
                                              sel_v, iota16, neg_inf), 0)
        pltpu.sync_copy(sel_v,
                        sel_hbm.at[pl.ds(b * E * TOPK + SEL_W * g, SEL_W)])


def _run_k2(ents, adj_rel_p, adj_nbr_p, g_mat, proj):
    mesh = plsc.VectorSubcoreMesh(core_axis_name="c", subcore_axis_name="s",
                                  num_cores=NC, num_subcores=NS)
    fn = pl.kernel(
        _sc_body,
        out_type=(jax.ShapeDtypeStruct((T, DIM), jnp.float32),
                  jax.ShapeDtypeStruct((T * TOPK,), jnp.int32)),
        mesh=mesh,
        compiler_params=pltpu.CompilerParams(needs_layout_passes=False),
        scratch_types=[
            pltpu.VMEM((128,), jnp.int32),
            pltpu.VMEM((16,), jnp.int32),
            pltpu.VMEM((16, 128), jnp.int32),
            pltpu.VMEM((16, 128), jnp.int32),
            pltpu.VMEM((E, 128), jnp.int32),
            pltpu.VMEM((E, DIM), jnp.float32),
            pltpu.VMEM((GROUP, NP), jnp.float32),
            pltpu.VMEM((GROUP, NP), jnp.float32),
            pltpu.VMEM((GROUP, DEG), jnp.float32),
            pltpu.VMEM((SEL_W,), jnp.int32),
            pltpu.SemaphoreType.DMA,
            pltpu.SemaphoreType.DMA,
            pltpu.SemaphoreType.DMA,
        ],
    )
    return fn(ents, adj_rel_p, adj_nbr_p, g_mat, proj)


def _layernorm(x, s, b):
    m = jnp.mean(x, axis=-1, keepdims=True)
    v = jnp.mean((x - m) ** 2, axis=-1, keepdims=True)
    return (x - m) / jnp.sqrt(v + EPS) * s + b


def _k3_body(states_ref, sel_ref, proj_ref, wv_ref, bv_ref, wo_ref, bo_ref,
             ln1s_ref, ln1b_ref, w1_ref, b1_ref, w2_ref, b2_ref,
             ln2s_ref, ln2b_ref, out_ref, counts_ref):
    riota = lax.broadcasted_iota(jnp.int32, (TBLK, NP), 1)
    for c in range(T // TBLK):
        rows = sel_ref[pl.ds(TBLK * c, TBLK), :]
        cnt = jnp.zeros((TBLK, NP), jnp.float32)
        for k in range(TOPK):
            cnt = cnt + (rows[:, k][:, None] == riota).astype(jnp.float32)
        counts_ref[pl.ds(TBLK * c, TBLK), :] = cnt
    ctx = jnp.dot(counts_ref[...], proj_ref[...],
                  preferred_element_type=jnp.float32) * jnp.float32(1.0 / TOPK)
    x = states_ref[...]
    for l in range(LAYERS):
        v = jnp.dot(ctx, wv_ref[l], preferred_element_type=jnp.float32) \
            + bv_ref[l]
        attn = jnp.dot(v, wo_ref[l], preferred_element_type=jnp.float32) \
            + bo_ref[l]
        x = _layernorm(x + attn, ln1s_ref[l], ln1b_ref[l])
        h = jnp.dot(x, w1_ref[l], preferred_element_type=jnp.float32) \
            + b1_ref[l]
        h = 0.5 * h * (1.0 + lax.erf(h * jnp.float32(0.7071067811865476)))
        ff = jnp.dot(h, w2_ref[l], preferred_element_type=jnp.float32) \
            + b2_ref[l]
        x = _layernorm(x + ff, ln2s_ref[l], ln2b_ref[l])
    out_ref[...] = x


def _run_k3(states0, sel2, proj, Wv, bv, Wo, bo, ln1_s, ln1_b,
            W1, b1, W2, b2, ln2_s, ln2_b):
    return pl.pallas_call(
        _k3_body,
        out_shape=jax.ShapeDtypeStruct((T, DIM), jnp.float32),
        scratch_shapes=[pltpu.VMEM((T, NP), jnp.float32)],
    )(states0, sel2, proj, Wv, bv.reshape(LAYERS, 1, DIM), Wo,
      bo.reshape(LAYERS, 1, DIM), ln1_s.reshape(LAYERS, 1, DIM),
      ln1_b.reshape(LAYERS, 1, DIM), W1, b1.reshape(LAYERS, 1, 4 * DIM),
      W2, b2.reshape(LAYERS, 1, DIM), ln2_s.reshape(LAYERS, 1, DIM),
      ln2_b.reshape(LAYERS, 1, DIM))


def kernel(triple_ids, adj_rel, adj_nbr, rel_semantics, W_proj, b_proj,
           Wq, bq, Wk, bk, Wv, bv, Wo, bo, ln1_s, ln1_b,
           W1, b1, W2, b2, ln2_s, ln2_b):
    del Wq, bq, Wk, bk
    adj_rel = adj_rel.astype(jnp.int32)
    adj_nbr = adj_nbr.astype(jnp.int32)
    proj, g_mat, adj_rel_p, adj_nbr_p = _run_k1(
        rel_semantics, W_proj, b_proj, adj_rel, adj_nbr)

    head = triple_ids[:, 0].astype(jnp.int32)
    tail = triple_ids[:, 2].astype(jnp.int32)
    ents = jnp.zeros((128,), jnp.int32).at[0:B].set(head).at[B:2 * B].set(tail)

    states0, sel_flat = _run_k2(ents, adj_rel_p, adj_nbr_p, g_mat, proj)
    sel2 = sel_flat.reshape(T, TOPK)
    out = _run_k3(states0, sel2, proj, Wv, bv, Wo, bo, ln1_s, ln1_b,
                  W1, b1, W2, b2, ln2_s, ln2_b)
    states = out.reshape(B, E, DIM)
    mask = jnp.ones((B, E), jnp.float32)
    return states, mask


# speedup vs baseline: 1.0215x; 1.0215x over previous
import jax
import jax.numpy as jnp
from jax import lax
from jax.experimental import pallas as pl
from jax.experimental.pallas import tpu as pltpu
from jax.experimental.pallas import tpu_sc as plsc

B = 32
DEG = 32
E = 2 * DEG
T = B * E
N_REL = 2000
NP = 2048
N_ENT = 10000
REL_DIM = 384
DIM = 256
TOPK = 8
LAYERS = 2
EPS = 1e-5
NC, NS = 2, 16
GROUP = 16
NGRP = E // GROUP
TBLK = 256
SEL_W = GROUP * TOPK


def _k1_body(rs_ref, wp_ref, bp_ref, proj_ref, g_ref):
    p = jnp.dot(rs_ref[...], wp_ref[...],
                preferred_element_type=jnp.float32) + bp_ref[...]
    p = jnp.concatenate([p, jnp.zeros((NP - N_REL, DIM), jnp.float32)], axis=0)
    proj_ref[...] = p
    norm = jnp.sqrt(jnp.sum(p * p, axis=-1, keepdims=True))
    nhat = p / jnp.maximum(norm, 1e-12)
    g_ref[...] = lax.dot_general(nhat, nhat, (((1,), (1,)), ((), ())),
                                 preferred_element_type=jnp.float32)


def _run_k1(rel_semantics, w_proj, b_proj):
    return pl.pallas_call(
        _k1_body,
        out_shape=(jax.ShapeDtypeStruct((NP, DIM), jnp.float32),
                   jax.ShapeDtypeStruct((NP, NP), jnp.float32)),
    )(rel_semantics, w_proj, b_proj.reshape(1, DIM))


def _sc_prep_body(ents_hbm, adj_rel_hbm, adj_nbr_hbm, rid_hbm, nrel_hbm,
                  ents_v, pidx, hrel, hnbr, nrel_v, sem_m):
    b = lax.axis_index("s") * NC + lax.axis_index("c")
    iota16 = lax.iota(jnp.int32, 16)
    pltpu.sync_copy(ents_hbm, ents_v)
    h_spl = plsc.load_gather(ents_v, [jnp.full((16,), b, jnp.int32)])
    t_spl = plsc.load_gather(ents_v, [jnp.full((16,), b + B, jnp.int32)])
    pidx[...] = jnp.where(iota16 == 0, h_spl, t_spl)
    cp_r = pltpu.async_copy(adj_rel_hbm.at[pidx], hrel, sem_m)
    cp_n = pltpu.async_copy(adj_nbr_hbm.at[pidx], hnbr, sem_m)
    cp_r.wait()
    cp_n.wait()
    cps = []
    for h in range(2):
        cps.append(pltpu.async_copy(adj_rel_hbm.at[hnbr.at[h, pl.ds(0, DEG)]],
                                    nrel_v.at[pl.ds(DEG * h, DEG)], sem_m))
    for cp in cps:
        cp.wait()
    for h in range(2):
        pltpu.sync_copy(hrel.at[h, pl.ds(0, DEG)],
                        rid_hbm.at[pl.ds(b * E + DEG * h, DEG)])
    pltpu.sync_copy(nrel_v, nrel_hbm.at[pl.ds(b * E, E)])


def _run_k2a(ents, adj_rel, adj_nbr):
    mesh = plsc.VectorSubcoreMesh(core_axis_name="c", subcore_axis_name="s",
                                  num_cores=NC, num_subcores=NS)
    fn = pl.kernel(
        _sc_prep_body,
        out_type=(jax.ShapeDtypeStruct((T,), jnp.int32),
                  jax.ShapeDtypeStruct((T, DEG), jnp.int32)),
        mesh=mesh,
        compiler_params=pltpu.CompilerParams(needs_layout_passes=False,
                                             use_tc_tiling_on_sc=False),
        scratch_types=[
            pltpu.VMEM((128,), jnp.int32),
            pltpu.VMEM((16,), jnp.int32),
            pltpu.VMEM((16, DEG), jnp.int32),
            pltpu.VMEM((16, DEG), jnp.int32),
            pltpu.VMEM((E, DEG), jnp.int32),
            pltpu.SemaphoreType.DMA,
        ],
    )
    return fn(ents, adj_rel, adj_nbr)


def _topk_step(j, base, nrel, sims, sel_ref, iota16, neg_inf):
    jj = j - base
    nid0 = nrel[j, pl.ds(0, 16)]
    nid1 = nrel[j, pl.ds(16, 16)]
    s0 = sims[jj, pl.ds(0, 16)]
    s1 = sims[jj, pl.ds(16, 16)]
    for it in range(TOPK):
        m = jnp.max(jnp.maximum(s0, s1))
        eq0 = s0 == m
        eq1 = s1 == m
        c0 = plsc.all_reduce_population_count(eq0)
        f0 = plsc.all_reduce_ffs(eq0)
        f1 = plsc.all_reduce_ffs(eq1)
        use0 = c0 > 0
        cand0 = nid0.at[f0].get(mode="promise_in_bounds")
        cand1 = nid1.at[f1].get(mode="promise_in_bounds")
        sel_id = jnp.where(use0, cand0, cand1)
        plsc.store_scatter(sel_ref,
                           [jnp.full((16,), jj * TOPK + it, jnp.int32)],
                           sel_id, mask=iota16 == 0)
        s0 = jnp.where(use0 & (iota16 == f0), neg_inf, s0)
        s1 = jnp.where((~use0) & (iota16 == f1), neg_inf, s1)
    return 0


def _sc_body(rid_hbm, nrel_hbm, g_hbm, proj_hbm,
             states0_hbm, sel_hbm,
             rid_v, nrel, relvec, ga, gb, sims, sel_v,
             sem_m, sem_ga, sem_gb):
    b = lax.axis_index("s") * NC + lax.axis_index("c")
    iota16 = lax.iota(jnp.int32, 16)
    neg_inf = jnp.float32(-jnp.inf)
    grows = (ga, gb)
    sem_g = (sem_ga, sem_gb)

    pltpu.sync_copy(rid_hbm.at[pl.ds(b * E, E)], rid_v)
    pltpu.async_copy(g_hbm.at[rid_v.at[pl.ds(0, GROUP)]], ga, sem_ga)
    cp_n = pltpu.async_copy(nrel_hbm.at[pl.ds(b * E, E)], nrel, sem_m)
    cp_v = pltpu.async_copy(proj_hbm.at[rid_v], relvec, sem_m)
    cp_n.wait()
    cp_v.wait()
    pltpu.sync_copy(relvec, states0_hbm.at[pl.ds(b * E, E)])

    for g in range(NGRP):
        cur = g % 2
        pltpu.make_async_copy(
            g_hbm.at[rid_v.at[pl.ds(GROUP * g, GROUP)]],
            grows[cur], sem_g[cur]).wait()
        if g + 1 < NGRP:
            g2 = g + 1
            pltpu.async_copy(
                g_hbm.at[rid_v.at[pl.ds(GROUP * g2, GROUP)]],
                grows[g2 % 2], sem_g[g2 % 2])
        for j in range(GROUP):
            e = GROUP * g + j
            rowj = jnp.full((16,), j, jnp.int32)
            sims[j, pl.ds(0, 16)] = plsc.load_gather(
                grows[cur], [rowj, nrel[e, pl.ds(0, 16)]])
            sims[j, pl.ds(16, 16)] = plsc.load_gather(
                grows[cur], [rowj, nrel[e, pl.ds(16, 16)]])
        base = GROUP * g
        lax.fori_loop(base, base + GROUP,
                      lambda j, u: _topk_step(j, base, nrel, sims,
                                              sel_v, iota16, neg_inf), 0)
        pltpu.sync_copy(sel_v,
                        sel_hbm.at[pl.ds(b * E * TOPK + SEL_W * g, SEL_W)])


def _run_k2(rid, nrel, g_mat, proj):
    mesh = plsc.VectorSubcoreMesh(core_axis_name="c", subcore_axis_name="s",
                                  num_cores=NC, num_subcores=NS)
    fn = pl.kernel(
        _sc_body,
        out_type=(jax.ShapeDtypeStruct((T, DIM), jnp.float32),
                  jax.ShapeDtypeStruct((T * TOPK,), jnp.int32)),
        mesh=mesh,
        compiler_params=pltpu.CompilerParams(needs_layout_passes=False),
        scratch_types=[
            pltpu.VMEM((E,), jnp.int32),
            pltpu.VMEM((E, DEG), jnp.int32),
            pltpu.VMEM((E, DIM), jnp.float32),
            pltpu.VMEM((GROUP, NP), jnp.float32),
            pltpu.VMEM((GROUP, NP), jnp.float32),
            pltpu.VMEM((GROUP, DEG), jnp.float32),
            pltpu.VMEM((SEL_W,), jnp.int32),
            pltpu.SemaphoreType.DMA,
            pltpu.SemaphoreType.DMA,
            pltpu.SemaphoreType.DMA,
        ],
    )
    return fn(rid, nrel, g_mat, proj)


def _layernorm(x, s, b):
    m = jnp.mean(x, axis=-1, keepdims=True)
    v = jnp.mean((x - m) ** 2, axis=-1, keepdims=True)
    return (x - m) / jnp.sqrt(v + EPS) * s + b


def _k3_body(states_ref, sel_ref, proj_ref, wv_ref, bv_ref, wo_ref, bo_ref,
             ln1s_ref, ln1b_ref, w1_ref, b1_ref, w2_ref, b2_ref,
             ln2s_ref, ln2b_ref, out_ref, counts_ref):
    riota = lax.broadcasted_iota(jnp.int32, (TBLK, NP), 1)
    for c in range(T // TBLK):
        rows = sel_ref[pl.ds(TBLK * c, TBLK), :]
        cnt = jnp.zeros((TBLK, NP), jnp.float32)
        for k in range(TOPK):
            cnt = cnt + (rows[:, k][:, None] == riota).astype(jnp.float32)
        counts_ref[pl.ds(TBLK * c, TBLK), :] = cnt
    ctx = jnp.dot(counts_ref[...], proj_ref[...],
                  preferred_element_type=jnp.float32) * jnp.float32(1.0 / TOPK)
    x = states_ref[...]
    for l in range(LAYERS):
        v = jnp.dot(ctx, wv_ref[l], preferred_element_type=jnp.float32) \
            + bv_ref[l]
        attn = jnp.dot(v, wo_ref[l], preferred_element_type=jnp.float32) \
            + bo_ref[l]
        x = _layernorm(x + attn, ln1s_ref[l], ln1b_ref[l])
        h = jnp.dot(x, w1_ref[l], preferred_element_type=jnp.float32) \
            + b1_ref[l]
        h = 0.5 * h * (1.0 + lax.erf(h * jnp.float32(0.7071067811865476)))
        ff = jnp.dot(h, w2_ref[l], preferred_element_type=jnp.float32) \
            + b2_ref[l]
        x = _layernorm(x + ff, ln2s_ref[l], ln2b_ref[l])
    out_ref[...] = x


def _run_k3(states0, sel2, proj, Wv, bv, Wo, bo, ln1_s, ln1_b,
            W1, b1, W2, b2, ln2_s, ln2_b):
    return pl.pallas_call(
        _k3_body,
        out_shape=jax.ShapeDtypeStruct((T, DIM), jnp.float32),
        scratch_shapes=[pltpu.VMEM((T, NP), jnp.float32)],
    )(states0, sel2, proj, Wv, bv.reshape(LAYERS, 1, DIM), Wo,
      bo.reshape(LAYERS, 1, DIM), ln1_s.reshape(LAYERS, 1, DIM),
      ln1_b.reshape(LAYERS, 1, DIM), W1, b1.reshape(LAYERS, 1, 4 * DIM),
      W2, b2.reshape(LAYERS, 1, DIM), ln2_s.reshape(LAYERS, 1, DIM),
      ln2_b.reshape(LAYERS, 1, DIM))


def kernel(triple_ids, adj_rel, adj_nbr, rel_semantics, W_proj, b_proj,
           Wq, bq, Wk, bk, Wv, bv, Wo, bo, ln1_s, ln1_b,
           W1, b1, W2, b2, ln2_s, ln2_b):
    del Wq, bq, Wk, bk
    adj_rel = adj_rel.astype(jnp.int32)
    adj_nbr = adj_nbr.astype(jnp.int32)
    head = triple_ids[:, 0].astype(jnp.int32)
    tail = triple_ids[:, 2].astype(jnp.int32)
    ents = jnp.zeros((128,), jnp.int32).at[0:B].set(head).at[B:2 * B].set(tail)
    rid, nrel = _run_k2a(ents, adj_rel, adj_nbr)
    proj, g_mat = _run_k1(rel_semantics, W_proj, b_proj)

    states0, sel_flat = _run_k2(rid, nrel, g_mat, proj)
    sel2 = sel_flat.reshape(T, TOPK)
    out = _run_k3(states0, sel2, proj, Wv, bv, Wo, bo, ln1_s, ln1_b,
                  W1, b1, W2, b2, ln2_s, ln2_b)
    states = out.reshape(B, E, DIM)
    mask = jnp.ones((B, E), jnp.float32)
    return states, mask
